# baseline (speedup 1.0000x reference)
import functools

import jax
import jax.numpy as jnp
from jax import lax
from jax.experimental import pallas as pl
from jax.experimental.pallas import tpu as pltpu

NUM_EXPERTS = 64
TILE = 512
NEG_INF = float("-inf")


def _router_kernel(x_ref, wt_ref, b_ref, rw_ref, se_ref):
    x = x_ref[...]
    wt = wt_ref[...]
    logits = jnp.dot(x, wt, preferred_element_type=jnp.float32)
    logits = logits + b_ref[...]

    t = logits.shape[0]
    iota = lax.broadcasted_iota(jnp.int32, (t, NUM_EXPERTS), 1)
    big = jnp.int32(NUM_EXPERTS)

    m1 = jnp.max(logits, axis=1, keepdims=True)
    i1 = jnp.min(jnp.where(logits == m1, iota, big), axis=1, keepdims=True)
    masked = jnp.where(iota == i1, NEG_INF, logits)
    m2 = jnp.max(masked, axis=1, keepdims=True)
    i2 = jnp.min(jnp.where(masked == m2, iota, big), axis=1, keepdims=True)

    w1 = jax.nn.sigmoid(m1 - m2)
    w2 = 1.0 - w1

    rw_ref[...] = jnp.concatenate([w1, w2], axis=1)
    se_ref[...] = jnp.concatenate([i1, i2], axis=1)


@functools.partial(jax.jit, static_argnames=())
def _run(x2d, wt, b2d):
    n_tokens = x2d.shape[0]
    d = x2d.shape[1]
    grid = (n_tokens // TILE,)
    rw, se = pl.pallas_call(
        _router_kernel,
        grid=grid,
        in_specs=[
            pl.BlockSpec((TILE, d), lambda i: (i, 0)),
            pl.BlockSpec((d, NUM_EXPERTS), lambda i: (0, 0)),
            pl.BlockSpec((1, NUM_EXPERTS), lambda i: (0, 0)),
        ],
        out_specs=[
            pl.BlockSpec((TILE, 2), lambda i: (i, 0)),
            pl.BlockSpec((TILE, 2), lambda i: (i, 0)),
        ],
        out_shape=[
            jax.ShapeDtypeStruct((n_tokens, 2), jnp.float32),
            jax.ShapeDtypeStruct((n_tokens, 2), jnp.int32),
        ],
        compiler_params=pltpu.CompilerParams(
            dimension_semantics=("parallel",),
        ),
    )(x2d, wt, b2d)
    return rw, se


def kernel(x, W, b):
    bsz, seq, d = x.shape
    x2d = x.reshape(bsz * seq, d)
    wt = W.T
    b2d = b.reshape(1, NUM_EXPERTS)
    rw, se = _run(x2d, wt, b2d)
    return rw.reshape(bsz, seq, 2), se.reshape(bsz, seq, 2)


# speedup vs baseline: 1.0582x; 1.0582x over previous
import functools

import jax
import jax.numpy as jnp
from jax import lax
from jax.experimental import pallas as pl
from jax.experimental.pallas import tpu as pltpu

NUM_EXPERTS = 64
TILE = 1024
NEG_INF = float("-inf")


def _router_kernel(x_ref, wt_ref, b_ref, rw_ref, se_ref):
    x = x_ref[...]
    wt = wt_ref[...]
    logits = jnp.dot(x, wt, preferred_element_type=jnp.float32)
    logits = logits + b_ref[...]

    t = logits.shape[0]
    iota = lax.broadcasted_iota(jnp.int32, (t, NUM_EXPERTS), 1)
    big = jnp.int32(NUM_EXPERTS)

    m1 = jnp.max(logits, axis=1, keepdims=True)
    i1 = jnp.min(jnp.where(logits == m1, iota, big), axis=1, keepdims=True)
    masked = jnp.where(iota == i1, NEG_INF, logits)
    m2 = jnp.max(masked, axis=1, keepdims=True)
    i2 = jnp.min(jnp.where(masked == m2, iota, big), axis=1, keepdims=True)

    w1 = jax.nn.sigmoid(m1 - m2)
    w2 = 1.0 - w1

    rw_ref[...] = jnp.concatenate([w1, w2], axis=1)
    se_ref[...] = jnp.concatenate([i1, i2], axis=1)


@functools.partial(jax.jit, static_argnames=())
def _run(x2d, wt, b2d):
    n_tokens = x2d.shape[0]
    d = x2d.shape[1]
    grid = (n_tokens // TILE,)
    rw, se = pl.pallas_call(
        _router_kernel,
        grid=grid,
        in_specs=[
            pl.BlockSpec((TILE, d), lambda i: (i, 0)),
            pl.BlockSpec((d, NUM_EXPERTS), lambda i: (0, 0)),
            pl.BlockSpec((1, NUM_EXPERTS), lambda i: (0, 0)),
        ],
        out_specs=[
            pl.BlockSpec((TILE, 2), lambda i: (i, 0)),
            pl.BlockSpec((TILE, 2), lambda i: (i, 0)),
        ],
        out_shape=[
            jax.ShapeDtypeStruct((n_tokens, 2), jnp.float32),
            jax.ShapeDtypeStruct((n_tokens, 2), jnp.int32),
        ],
        compiler_params=pltpu.CompilerParams(
            dimension_semantics=("parallel",),
        ),
    )(x2d, wt, b2d)
    return rw, se


def kernel(x, W, b):
    bsz, seq, d = x.shape
    x2d = x.reshape(bsz * seq, d)
    wt = W.T
    b2d = b.reshape(1, NUM_EXPERTS)
    rw, se = _run(x2d, wt, b2d)
    return rw.reshape(bsz, seq, 2), se.reshape(bsz, seq, 2)


# speedup vs baseline: 1.0951x; 1.0349x over previous
import functools

import jax
import jax.numpy as jnp
from jax import lax
from jax.experimental import pallas as pl
from jax.experimental.pallas import tpu as pltpu

NUM_EXPERTS = 64
TILE = 1024
NEG_INF = float("-inf")


def _router_kernel(x_ref, w_ref, b_ref, rw_ref, se_ref):
    x = x_ref[...]
    w = w_ref[...]
    logits = lax.dot_general(
        x, w,
        dimension_numbers=(((1,), (1,)), ((), ())),
        preferred_element_type=jnp.float32,
    )
    logits = logits + b_ref[...]

    t = logits.shape[0]
    iota = lax.broadcasted_iota(jnp.int32, (t, NUM_EXPERTS), 1)
    big = jnp.int32(NUM_EXPERTS)

    m1 = jnp.max(logits, axis=1, keepdims=True)
    i1 = jnp.min(jnp.where(logits == m1, iota, big), axis=1, keepdims=True)
    masked = jnp.where(iota == i1, NEG_INF, logits)
    m2 = jnp.max(masked, axis=1, keepdims=True)
    i2 = jnp.min(jnp.where(masked == m2, iota, big), axis=1, keepdims=True)

    w1 = jax.nn.sigmoid(m1 - m2)
    w2 = 1.0 - w1

    rw_ref[...] = jnp.concatenate([w1, w2], axis=1)
    se_ref[...] = jnp.concatenate([i1, i2], axis=1)


@functools.partial(jax.jit, static_argnames=())
def _run(x2d, W, b2d):
    n_tokens = x2d.shape[0]
    d = x2d.shape[1]
    grid = (n_tokens // TILE,)
    rw, se = pl.pallas_call(
        _router_kernel,
        grid=grid,
        in_specs=[
            pl.BlockSpec((TILE, d), lambda i: (i, 0)),
            pl.BlockSpec((NUM_EXPERTS, d), lambda i: (0, 0)),
            pl.BlockSpec((1, NUM_EXPERTS), lambda i: (0, 0)),
        ],
        out_specs=[
            pl.BlockSpec((TILE, 2), lambda i: (i, 0)),
            pl.BlockSpec((TILE, 2), lambda i: (i, 0)),
        ],
        out_shape=[
            jax.ShapeDtypeStruct((n_tokens, 2), jnp.float32),
            jax.ShapeDtypeStruct((n_tokens, 2), jnp.int32),
        ],
        compiler_params=pltpu.CompilerParams(
            dimension_semantics=("parallel",),
        ),
    )(x2d, W, b2d)
    return rw, se


def kernel(x, W, b):
    bsz, seq, d = x.shape
    x2d = x.reshape(bsz * seq, d)
    b2d = b.reshape(1, NUM_EXPERTS)
    rw, se = _run(x2d, W, b2d)
    return rw.reshape(bsz, seq, 2), se.reshape(bsz, seq, 2)


# speedup vs baseline: 1.1005x; 1.0050x over previous
import functools

import jax
import jax.numpy as jnp
from jax import lax
from jax.experimental import pallas as pl
from jax.experimental.pallas import tpu as pltpu

NUM_EXPERTS = 64
TILE = 1024
NEG_INF = float("-inf")


def _router_kernel(x_ref, w_ref, b_ref, rw_ref, se_ref):
    x = x_ref[0]
    w = w_ref[...]
    logits = lax.dot_general(
        x, w,
        dimension_numbers=(((1,), (1,)), ((), ())),
        preferred_element_type=jnp.float32,
    )
    logits = logits + b_ref[...]

    t = logits.shape[0]
    iota = lax.broadcasted_iota(jnp.int32, (t, NUM_EXPERTS), 1)
    big = jnp.int32(NUM_EXPERTS)

    m1 = jnp.max(logits, axis=1, keepdims=True)
    i1 = jnp.min(jnp.where(logits == m1, iota, big), axis=1, keepdims=True)
    masked = jnp.where(iota == i1, NEG_INF, logits)
    m2 = jnp.max(masked, axis=1, keepdims=True)
    i2 = jnp.min(jnp.where(masked == m2, iota, big), axis=1, keepdims=True)

    w1 = jax.nn.sigmoid(m1 - m2)
    w2 = 1.0 - w1

    rw_ref[0] = jnp.concatenate([w1, w2], axis=1)
    se_ref[0] = jnp.concatenate([i1, i2], axis=1)


@functools.partial(jax.jit, static_argnames=())
def _run(x, W, b2d):
    bsz, seq, d = x.shape
    grid = (bsz, seq // TILE)
    rw, se = pl.pallas_call(
        _router_kernel,
        grid=grid,
        in_specs=[
            pl.BlockSpec((1, TILE, d), lambda bi, i: (bi, i, 0)),
            pl.BlockSpec((NUM_EXPERTS, d), lambda bi, i: (0, 0)),
            pl.BlockSpec((1, NUM_EXPERTS), lambda bi, i: (0, 0)),
        ],
        out_specs=[
            pl.BlockSpec((1, TILE, 2), lambda bi, i: (bi, i, 0)),
            pl.BlockSpec((1, TILE, 2), lambda bi, i: (bi, i, 0)),
        ],
        out_shape=[
            jax.ShapeDtypeStruct((bsz, seq, 2), jnp.float32),
            jax.ShapeDtypeStruct((bsz, seq, 2), jnp.int32),
        ],
        compiler_params=pltpu.CompilerParams(
            dimension_semantics=("parallel", "parallel"),
        ),
    )(x, W, b2d)
    return rw, se


def kernel(x, W, b):
    b2d = b.reshape(1, NUM_EXPERTS)
    return _run(x, W, b2d)
